# baseline (speedup 1.0000x reference)
import functools

import jax
import jax.numpy as jnp
from jax import lax
from jax.experimental import pallas as pl
from jax.experimental.pallas import tpu as pltpu
from jax.experimental.pallas import tpu_sc as plsc

BATCH = 4
SEQLEN = 2048
EMBED = 64
NUM_CORES = 2
NUM_SUBCORES = 16
NW = NUM_CORES * NUM_SUBCORES
TOTAL = BATCH * SEQLEN
CHUNK = TOTAL // NW
GCH = 128
NG = CHUNK // GCH
LANES = 16


@functools.partial(
    pl.kernel,
    out_type=jax.ShapeDtypeStruct((TOTAL, EMBED), jnp.float32),
    mesh=plsc.VectorSubcoreMesh(core_axis_name="c", subcore_axis_name="s"),
    scratch_types=[
        pltpu.VMEM((NG, GCH), jnp.int32),
        pltpu.VMEM((CHUNK, EMBED), jnp.float32),
        pltpu.VMEM((CHUNK, EMBED), jnp.float32),
        pltpu.SemaphoreType.DMA,
    ],
    compiler_params=pltpu.CompilerParams(use_tc_tiling_on_sc=False),
)
def _embed_lookup(x_hbm, tok_hbm, pos_hbm, out_hbm, idx_v, rows_v, pos_v, sem):
    wid = lax.axis_index("s") * NUM_CORES + lax.axis_index("c")
    base = wid * CHUNK
    pos_base = lax.rem(base, SEQLEN)

    pltpu.sync_copy(x_hbm.at[pl.ds(wid * NG, NG)], idx_v)
    pos_cp = pltpu.async_copy(pos_hbm.at[pl.ds(pos_base, CHUNK)], pos_v, sem)

    gathers = [
        pltpu.async_copy(
            tok_hbm.at[idx_v.at[j]], rows_v.at[pl.ds(j * GCH, GCH)], sem
        )
        for j in range(NG)
    ]
    pos_cp.wait()
    for cp in gathers:
        cp.wait()

    def body(i, carry):
        for c in range(EMBED // LANES):
            sl = pl.ds(c * LANES, LANES)
            plsc.addupdate(rows_v.at[i, sl], pos_v[i, sl])
        return carry

    lax.fori_loop(0, CHUNK, body, 0)

    pltpu.sync_copy(rows_v, out_hbm.at[pl.ds(base, CHUNK)])


def kernel(x, token_table, pos_table):
    x2 = x.reshape(NW * NG, GCH)
    out = _embed_lookup(x2, token_table, pos_table)
    return out.reshape(BATCH, SEQLEN, EMBED).astype(jnp.bfloat16)
